# baseline (speedup 1.0000x reference)
import functools

import jax
import jax.numpy as jnp
from jax import lax
from jax.experimental import pallas as pl
from jax.experimental.pallas import tpu as pltpu
from jax.experimental.pallas import tpu_sc as plsc

_NE = 8
_R = 64
_SCALING = 32.0 / 64.0
_BMG = 512
_BM = 2048
_BN = 256
_NEG = -1e30
_NSC = 32
_LANES = 16


def _gate_body(x_ref, wg_ref, aall_ref, b_ref, x16_ref, h_ref, logt_ref,
               bcat_ref, *, ne):
    m = pl.program_id(0)

    @pl.when(m < ne)
    def _bcat():
        bcat_ref[...] = jnp.transpose(
            b_ref[0], (1, 0)).astype(jnp.bfloat16)

    xb = x_ref[...].astype(jnp.bfloat16)
    x16_ref[...] = xb
    logt_ref[...] = jax.lax.dot_general(
        wg_ref[...].astype(jnp.bfloat16), xb, (((1,), (1,)), ((), ())),
        preferred_element_type=jnp.float32)
    h = jax.lax.dot_general(
        xb, aall_ref[...].astype(jnp.bfloat16), (((1,), (1,)), ((), ())),
        preferred_element_type=jnp.float32)
    h_ref[...] = (h * _SCALING).astype(jnp.bfloat16)


def _route_body(logt_hbm, wt_hbm, slab, wslab, *, ne, tok_per_sc):
    wid = lax.axis_index("s") * 2 + lax.axis_index("c")
    base = wid * tok_per_sc
    pltpu.sync_copy(logt_hbm.at[:, pl.ds(base, tok_per_sc)], slab)
    for c in range(tok_per_sc // _LANES):
        sl = pl.ds(c * _LANES, _LANES)
        l = [slab[e, sl] for e in range(ne)]
        m1 = l[0]
        for e in range(1, ne):
            m1 = jnp.maximum(m1, l[e])
        taken = l[0] - l[0]
        oh1 = []
        for e in range(ne):
            hit = jnp.where(l[e] == m1, 1.0, 0.0) * (1.0 - taken)
            oh1.append(hit)
            taken = jnp.maximum(taken, hit)
        m2 = None
        masked = []
        for e in range(ne):
            me = l[e] + oh1[e] * _NEG
            masked.append(me)
            m2 = me if m2 is None else jnp.maximum(m2, me)
        p1 = 1.0 / (1.0 + jnp.exp(m2 - m1))
        p2 = 1.0 - p1
        taken2 = taken - taken
        for e in range(ne):
            hit2 = jnp.where(masked[e] == m2, 1.0, 0.0) * (1.0 - taken2)
            taken2 = jnp.maximum(taken2, hit2)
            wslab[e, sl] = oh1[e] * p1 + hit2 * p2
    pltpu.sync_copy(wslab, wt_hbm.at[:, pl.ds(base, tok_per_sc)])


def _mm_body(x16_ref, wb_ref, hs_ref, wt_ref, bcat_ref, out_ref, *, ne, r):
    n = pl.program_id(1)

    @pl.when(n == 0)
    def _scale_h():
        ner = ne * r
        col_e = jax.lax.broadcasted_iota(jnp.int32, (ne, ner), 1) // r
        row_e = jax.lax.broadcasted_iota(jnp.int32, (ne, ner), 0)
        expand = (col_e == row_e).astype(jnp.float32)
        wexp = jax.lax.dot_general(
            wt_ref[...], expand, (((0,), (0,)), ((), ())),
            preferred_element_type=jnp.float32)
        hs_ref[...] = (hs_ref[...].astype(jnp.float32) * wexp
                       ).astype(jnp.bfloat16)

    acc = jax.lax.dot_general(
        x16_ref[...], wb_ref[...].astype(jnp.bfloat16),
        (((1,), (1,)), ((), ())),
        preferred_element_type=jnp.float32)
    acc += jnp.dot(hs_ref[...], bcat_ref[...],
                   preferred_element_type=jnp.float32)
    out_ref[...] = acc


@functools.partial(jax.jit, static_argnames=("bmg", "bm", "bn", "interpret"))
def _mixlora(x, wb, wg, aall, b, bmg=_BMG, bm=_BM, bn=_BN,
             interpret=False):
    ntok, d = x.shape
    ne = wg.shape[0]
    ner = aall.shape[0]
    r = ner // ne
    x16, h16, logt, bcat = pl.pallas_call(
        functools.partial(_gate_body, ne=ne),
        grid=(ntok // bmg,),
        in_specs=[
            pl.BlockSpec((bmg, d), lambda m: (m, 0)),
            pl.BlockSpec((ne, d), lambda m: (0, 0)),
            pl.BlockSpec((ner, d), lambda m: (0, 0)),
            pl.BlockSpec((1, d, r),
                         lambda m: (jnp.minimum(m, ne - 1), 0, 0)),
        ],
        out_specs=[
            pl.BlockSpec((bmg, d), lambda m: (m, 0)),
            pl.BlockSpec((bmg, ner), lambda m: (m, 0)),
            pl.BlockSpec((ne, bmg), lambda m: (0, m)),
            pl.BlockSpec((r, d),
                         lambda m: (jnp.minimum(m, ne - 1), 0)),
        ],
        out_shape=[
            jax.ShapeDtypeStruct((ntok, d), jnp.bfloat16),
            jax.ShapeDtypeStruct((ntok, ner), jnp.bfloat16),
            jax.ShapeDtypeStruct((ne, ntok), jnp.float32),
            jax.ShapeDtypeStruct((ner, d), jnp.bfloat16),
        ],
        compiler_params=pltpu.CompilerParams(
            dimension_semantics=("arbitrary",)),
        interpret=interpret,
    )(x, wg, aall, b)

    tok_per_sc = ntok // _NSC
    mesh = plsc.VectorSubcoreMesh(core_axis_name="c", subcore_axis_name="s")
    wt = pl.kernel(
        functools.partial(_route_body, ne=ne, tok_per_sc=tok_per_sc),
        out_type=jax.ShapeDtypeStruct((ne, ntok), jnp.float32),
        mesh=mesh,
        scratch_types=[
            pltpu.VMEM((ne, tok_per_sc), jnp.float32),
            pltpu.VMEM((ne, tok_per_sc), jnp.float32),
        ],
    )(logt)

    return pl.pallas_call(
        functools.partial(_mm_body, ne=ne, r=r),
        grid=(ntok // bm, d // bn),
        in_specs=[
            pl.BlockSpec((bm, d), lambda m, n: (m, 0)),
            pl.BlockSpec((bn, d), lambda m, n: (n, 0)),
            pl.BlockSpec((bm, ner), lambda m, n: (m, 0),
                         pipeline_mode=pl.Buffered(buffer_count=1)),
            pl.BlockSpec((ne, bm), lambda m, n: (0, m)),
            pl.BlockSpec((ner, bn), lambda m, n: (0, n)),
        ],
        out_specs=pl.BlockSpec((bm, bn), lambda m, n: (m, n)),
        out_shape=jax.ShapeDtypeStruct((ntok, d), jnp.float32),
        compiler_params=pltpu.CompilerParams(
            dimension_semantics=("arbitrary", "arbitrary")),
        interpret=interpret,
    )(x16, wb, h16, wt, bcat)


def kernel(x, W_base, W_gate, A, B):
    ne, r, d = A.shape
    aall = A.reshape(ne * r, d)
    return _mixlora(x, W_base, W_gate, aall, B)


# speedup vs baseline: 1.0357x; 1.0357x over previous
import functools

import jax
import jax.numpy as jnp
from jax import lax
from jax.experimental import pallas as pl
from jax.experimental.pallas import tpu as pltpu
from jax.experimental.pallas import tpu_sc as plsc

_NE = 8
_R = 64
_SCALING = 32.0 / 64.0
_BMG = 512
_BM = 2048
_BN = 256
_NEG = -1e30
_NSC = 32
_LANES = 16


def _gate_body(x_ref, wg_ref, aall_ref, x16_ref, h_ref, logt_ref):
    xb = x_ref[...].astype(jnp.bfloat16)
    x16_ref[...] = xb
    logt_ref[...] = jax.lax.dot_general(
        wg_ref[...].astype(jnp.bfloat16), xb, (((1,), (1,)), ((), ())),
        preferred_element_type=jnp.float32)
    h = jax.lax.dot_general(
        xb, aall_ref[...].astype(jnp.bfloat16), (((1,), (1,)), ((), ())),
        preferred_element_type=jnp.float32)
    h_ref[...] = (h * _SCALING).astype(jnp.bfloat16)


def _route_body(logt_hbm, wt_hbm, slab, wslab, *, ne, tok_per_sc):
    wid = lax.axis_index("s") * 2 + lax.axis_index("c")
    base = wid * tok_per_sc
    pltpu.sync_copy(logt_hbm.at[:, pl.ds(base, tok_per_sc)], slab)
    for c in range(tok_per_sc // _LANES):
        sl = pl.ds(c * _LANES, _LANES)
        l = [slab[e, sl] for e in range(ne)]
        m1 = l[0]
        for e in range(1, ne):
            m1 = jnp.maximum(m1, l[e])
        taken = l[0] - l[0]
        oh1 = []
        for e in range(ne):
            hit = jnp.where(l[e] == m1, 1.0, 0.0) * (1.0 - taken)
            oh1.append(hit)
            taken = jnp.maximum(taken, hit)
        m2 = None
        masked = []
        for e in range(ne):
            me = l[e] + oh1[e] * _NEG
            masked.append(me)
            m2 = me if m2 is None else jnp.maximum(m2, me)
        p1 = 1.0 / (1.0 + jnp.exp(m2 - m1))
        p2 = 1.0 - p1
        taken2 = taken - taken
        for e in range(ne):
            hit2 = jnp.where(masked[e] == m2, 1.0, 0.0) * (1.0 - taken2)
            taken2 = jnp.maximum(taken2, hit2)
            wslab[e, sl] = oh1[e] * p1 + hit2 * p2
    pltpu.sync_copy(wslab, wt_hbm.at[:, pl.ds(base, tok_per_sc)])


def _mm_body(x16_ref, wb_ref, hs_ref, wt_ref, bcat_ref, out_ref, *, ne, r):
    n = pl.program_id(1)

    @pl.when(n == 0)
    def _scale_h():
        ner = ne * r
        col_e = jax.lax.broadcasted_iota(jnp.int32, (ne, ner), 1) // r
        row_e = jax.lax.broadcasted_iota(jnp.int32, (ne, ner), 0)
        expand = (col_e == row_e).astype(jnp.float32)
        wexp = jax.lax.dot_general(
            wt_ref[...], expand, (((0,), (0,)), ((), ())),
            preferred_element_type=jnp.float32)
        hs_ref[...] = (hs_ref[...].astype(jnp.float32) * wexp
                       ).astype(jnp.bfloat16)

    acc = jax.lax.dot_general(
        x16_ref[...], wb_ref[...].astype(jnp.bfloat16),
        (((1,), (1,)), ((), ())),
        preferred_element_type=jnp.float32)
    acc += jnp.dot(hs_ref[...], bcat_ref[...],
                   preferred_element_type=jnp.float32)
    out_ref[...] = acc


@functools.partial(jax.jit, static_argnames=("bmg", "bm", "bn", "interpret"))
def _mixlora(x, wb, wg, aall, bcat, bmg=_BMG, bm=_BM, bn=_BN,
             interpret=False):
    ntok, d = x.shape
    ne = wg.shape[0]
    ner = aall.shape[0]
    r = ner // ne
    x16, h16, logt = pl.pallas_call(
        _gate_body,
        grid=(ntok // bmg,),
        in_specs=[
            pl.BlockSpec((bmg, d), lambda m: (m, 0)),
            pl.BlockSpec((ne, d), lambda m: (0, 0)),
            pl.BlockSpec((ner, d), lambda m: (0, 0)),
        ],
        out_specs=[
            pl.BlockSpec((bmg, d), lambda m: (m, 0)),
            pl.BlockSpec((bmg, ner), lambda m: (m, 0)),
            pl.BlockSpec((ne, bmg), lambda m: (0, m)),
        ],
        out_shape=[
            jax.ShapeDtypeStruct((ntok, d), jnp.bfloat16),
            jax.ShapeDtypeStruct((ntok, ner), jnp.bfloat16),
            jax.ShapeDtypeStruct((ne, ntok), jnp.float32),
        ],
        compiler_params=pltpu.CompilerParams(
            dimension_semantics=("arbitrary",)),
        interpret=interpret,
    )(x, wg, aall)

    tok_per_sc = ntok // _NSC
    mesh = plsc.VectorSubcoreMesh(core_axis_name="c", subcore_axis_name="s")
    wt = pl.kernel(
        functools.partial(_route_body, ne=ne, tok_per_sc=tok_per_sc),
        out_type=jax.ShapeDtypeStruct((ne, ntok), jnp.float32),
        mesh=mesh,
        scratch_types=[
            pltpu.VMEM((ne, tok_per_sc), jnp.float32),
            pltpu.VMEM((ne, tok_per_sc), jnp.float32),
        ],
    )(logt)

    return pl.pallas_call(
        functools.partial(_mm_body, ne=ne, r=r),
        grid=(ntok // bm, d // bn),
        in_specs=[
            pl.BlockSpec((bm, d), lambda m, n: (m, 0)),
            pl.BlockSpec((bn, d), lambda m, n: (n, 0)),
            pl.BlockSpec((bm, ner), lambda m, n: (m, 0),
                         pipeline_mode=pl.Buffered(buffer_count=1)),
            pl.BlockSpec((ne, bm), lambda m, n: (0, m)),
            pl.BlockSpec((ner, bn), lambda m, n: (0, n)),
        ],
        out_specs=pl.BlockSpec((bm, bn), lambda m, n: (m, n)),
        out_shape=jax.ShapeDtypeStruct((ntok, d), jnp.float32),
        compiler_params=pltpu.CompilerParams(
            dimension_semantics=("arbitrary", "arbitrary")),
        interpret=interpret,
    )(x16, wb, h16, wt, bcat)


def kernel(x, W_base, W_gate, A, B):
    ne, r, d = A.shape
    aall = A.reshape(ne * r, d)
    bcat = B.transpose(0, 2, 1).reshape(ne * r, d).astype(jnp.bfloat16)
    return _mixlora(x, W_base, W_gate, aall, bcat)
